# baseline (speedup 1.0000x reference)
import jax
import jax.numpy as jnp
from jax.experimental import pallas as pl
from jax.experimental.pallas import tpu as pltpu


def _dot(a, b):
    return jax.lax.dot_general(a, b, (((1,), (0,)), ((), ())),
                               preferred_element_type=jnp.float32)


def _dot_tn(a, b):
    return jax.lax.dot_general(a, b, (((0,), (0,)), ((), ())),
                               preferred_element_type=jnp.float32)


def _dot_nt(a, b):
    return jax.lax.dot_general(a, b, (((1,), (1,)), ((), ())),
                               preferred_element_type=jnp.float32)


def _body(x_ref, adj_s_ref, adj_t_ref, fcw_ref, w_ref, out_ref):
    x = x_ref[...]
    adj_t = adj_t_ref[...]
    adj_s = adj_s_ref[...]

    tw1t = w_ref[0:8, 0:512]
    tw3 = w_ref[8:12, 0:512]
    sw1t = w_ref[16:24, :]
    sw3 = w_ref[24:28, :]
    tw2 = w_ref[32:40, 0:4]
    sw2 = w_ref[40:48, 0:4]
    d1w = w_ref[48:50, 0:8]
    d1b = w_ref[56:57, 0:8]
    d2w = w_ref[64:72, 0:4]
    d2b = w_ref[72:73, 0:4]
    d3w = w_ref[80:84, 0:1]
    d3b = w_ref[88:89, 0:1]
    fcb = w_ref[96:97, 0:512]

    t1 = _dot_nt(tw1t, x)
    h = jnp.maximum(_dot_nt(t1, adj_t), 0.0)
    h = jnp.maximum(_dot_nt(_dot_tn(tw2, h), adj_t), 0.0)
    r = _dot_nt(h, adj_t)
    x_t = jnp.maximum(_dot_tn(r, tw3), 0.0)

    s1 = _dot(sw1t, x)
    g = jnp.maximum(_dot_nt(s1, adj_s), 0.0)
    g = jnp.maximum(_dot_nt(_dot_tn(sw2, g), adj_s), 0.0)
    q = _dot_nt(g, adj_s)
    x_st = jnp.maximum(_dot_tn(sw3, q), 0.0)

    m = _dot(d1w, _dot(d2w, d3w))
    b_eff = _dot(_dot(d1b, d2w) + d2b, d3w) + d3b

    fused = m[0, 0] * x_st + m[1, 0] * x_t + b_eff[0, 0]

    out_ref[...] = _dot_nt(fused, fcw_ref[...]) + fcb


def _pad8(a, width):
    r, c = a.shape
    return jnp.pad(a, ((0, (-r) % 8), (0, width - c)))


def kernel(x, x_adj_s, x_adj_t, t_W1, t_W2, t_W3, s_W1, s_W2, s_W3,
           dec1_W, dec1_b, dec2_W, dec2_b, dec3_W, dec3_b, fc_W, fc_b):
    T, Kd = x.shape
    f32 = jnp.float32

    pieces = [t_W1[0].T, t_W3[0], s_W1[0].T, s_W3[0], t_W2[0], s_W2[0],
              dec1_W, dec1_b[None, :], dec2_W, dec2_b[None, :],
              dec3_W, dec3_b[None, :], fc_b[None, :]]
    packed = jnp.concatenate([_pad8(p.astype(f32), T) for p in pieces], axis=0)

    vmem = pl.BlockSpec(memory_space=pltpu.VMEM)
    out = pl.pallas_call(
        _body,
        out_shape=jax.ShapeDtypeStruct((T, Kd), f32),
        in_specs=[vmem] * 5,
        out_specs=vmem,
    )(x, x_adj_s, x_adj_t, fc_W, packed)
    return out
